# baseline (speedup 1.0000x reference)
import functools

import jax
import jax.numpy as jnp
from jax import lax
from jax.experimental import pallas as pl
from jax.experimental.pallas import tpu as pltpu
from jax.experimental.pallas import tpu_sc as plsc

N = 10000
E = 320000
F = 128
D = 256
C = 10
G = 128

NPAD = 10240
HALF = 128
E_PAD = 327680
TR_E = 80
TR_F = 160
CH_F = 32
NCH_F = TR_F // CH_F
CH_E = 16
NCH_E = TR_E // CH_E
ROWS_PER_TILE = NPAD // 16
ZCHUNKS = ROWS_PER_TILE // 128
DUMP = N

def _make_sc_body(ch, nch):
    def body(tab_hbm, src_hbm, dst_hbm, z_hbm, out_hbm,
             src_v, dst_v, rows0, rows1, acc, sg0, sg1, ss0, ss1):
        c = lax.axis_index("c")
        s = lax.axis_index("s")
        wid = c * 16 + s

        def wait(sem, buf):
            pltpu.make_async_copy(z_hbm, buf, sem).wait()

        pltpu.sync_copy(z_hbm, rows0)
        for k in range(ZCHUNKS):
            pltpu.sync_copy(rows0, acc.at[pl.ds((s * ZCHUNKS + k) * 128, 128)])
        plsc.subcore_barrier()

        def chunk(ci, carry):
            pltpu.sync_copy(src_hbm.at[wid, pl.ds(ci * ch, ch)], src_v)
            pltpu.sync_copy(dst_hbm.at[wid, pl.ds(ci * ch, ch)], dst_v)
            pltpu.async_copy(tab_hbm.at[src_v.at[0]], rows0, sg0)

            def pair(j2, carry2):
                j = 2 * j2
                wait(sg0, rows0)
                pltpu.async_copy(rows0, acc.at[dst_v.at[j]], ss0, add=True)

                @pl.when(j2 >= 1)
                def _():
                    wait(ss1, rows1)

                pltpu.async_copy(tab_hbm.at[src_v.at[j + 1]], rows1, sg1)
                wait(sg1, rows1)
                pltpu.async_copy(rows1, acc.at[dst_v.at[j + 1]], ss1, add=True)
                wait(ss0, rows0)

                @pl.when(j + 2 < ch)
                def _():
                    pltpu.async_copy(tab_hbm.at[src_v.at[j + 2]], rows0, sg0)

                return carry2

            lax.fori_loop(0, ch // 2, pair, carry, unroll=False)
            wait(ss1, rows1)
            return carry

        lax.fori_loop(0, nch, chunk, 0, unroll=False)
        plsc.subcore_barrier()
        pltpu.sync_copy(acc.at[pl.ds(s * ROWS_PER_TILE, ROWS_PER_TILE)],
                        out_hbm.at[c, pl.ds(s * ROWS_PER_TILE, ROWS_PER_TILE)])

    return body


@functools.lru_cache(maxsize=1)
def _get_sc_kernels():
    mesh = plsc.VectorSubcoreMesh(core_axis_name="c", subcore_axis_name="s",
                                  num_cores=2, num_subcores=16)

    def build(ch, nch):
        return pl.kernel(
            _make_sc_body(ch, nch),
            out_type=jax.ShapeDtypeStruct((2, NPAD, HALF), jnp.float32),
            mesh=mesh,
            scratch_types=[
                pltpu.VMEM((ch, 128), jnp.int32),
                pltpu.VMEM((ch, 128), jnp.int32),
                pltpu.VMEM((128, HALF), jnp.float32),
                pltpu.VMEM((128, HALF), jnp.float32),
                pltpu.VMEM_SHARED((NPAD, HALF), jnp.float32),
                pltpu.SemaphoreType.DMA,
                pltpu.SemaphoreType.DMA,
                pltpu.SemaphoreType.DMA,
                pltpu.SemaphoreType.DMA,
            ],
        )

    return build(CH_E, NCH_E), build(CH_F, NCH_F)


BN = 1024
NB = NPAD // BN


def _mlp1_body(x_ref, p_ref, A_ref, c_ref, W2_ref, b2_ref, o_ref):
    u = x_ref[...] + p_ref[0] + p_ref[1]
    y = jnp.dot(u, A_ref[...], preferred_element_type=jnp.float32) + c_ref[...]
    y = jnp.maximum(y, 0.0)
    z = jnp.dot(y, W2_ref[...], preferred_element_type=jnp.float32) + b2_ref[...]
    z = jnp.maximum(z, 0.0)
    o_ref[0] = z[:, :HALF]
    o_ref[1] = z[:, HALF:]


def _mlpN_body(h_ref, a_ref, A_ref, c_ref, W2_ref, b2_ref, o_ref):
    u = jnp.concatenate([h_ref[0] + a_ref[0], h_ref[1] + a_ref[1]], axis=1)
    y = jnp.dot(u, A_ref[...], preferred_element_type=jnp.float32) + c_ref[...]
    y = jnp.maximum(y, 0.0)
    z = jnp.dot(y, W2_ref[...], preferred_element_type=jnp.float32) + b2_ref[...]
    z = jnp.maximum(z, 0.0)
    o_ref[0] = z[:, :HALF]
    o_ref[1] = z[:, HALF:]


_full = lambda i: (0, 0)
_rows1 = pl.BlockSpec((BN, HALF), lambda i: (i, 0))
_rows2 = pl.BlockSpec((2, BN, HALF), lambda i: (0, i, 0))
_wspec = pl.BlockSpec((D, D), _full)
_w1spec = pl.BlockSpec((F, D), _full)
_bspec = pl.BlockSpec((1, D), _full)

_mlp1 = pl.pallas_call(
    _mlp1_body,
    grid=(NB,),
    in_specs=[_rows1, _rows2, _w1spec, _bspec, _wspec, _bspec],
    out_specs=_rows2,
    out_shape=jax.ShapeDtypeStruct((2, NPAD, HALF), jnp.float32),
    compiler_params=pltpu.CompilerParams(
        dimension_semantics=("parallel",)),
)

_mlpN = pl.pallas_call(
    _mlpN_body,
    grid=(NB,),
    in_specs=[_rows2, _rows2, _wspec, _bspec, _wspec, _bspec],
    out_specs=_rows2,
    out_shape=jax.ShapeDtypeStruct((2, NPAD, HALF), jnp.float32),
    compiler_params=pltpu.CompilerParams(
        dimension_semantics=("parallel",)),
)


def _pool_body(h_ref, b_ref, L1_ref, lb1_ref, L2_ref, lb2_ref, o_ref, hg_acc):
    i = pl.program_id(0)

    @pl.when(i == 0)
    def _():
        hg_acc[...] = jnp.zeros_like(hg_acc)

    b = b_ref[0, 0, :]
    oneh = (b[None, :] == lax.broadcasted_iota(jnp.int32, (G, BN), 0)
            ).astype(jnp.float32)
    hrow = jnp.concatenate([h_ref[0], h_ref[1]], axis=1)
    hg_acc[...] += jnp.dot(oneh, hrow, preferred_element_type=jnp.float32)

    @pl.when(i == NB - 1)
    def _():
        hg = hg_acc[...]
        t = jnp.dot(hg, L1_ref[...], preferred_element_type=jnp.float32)
        t = jnp.maximum(t + lb1_ref[...], 0.0)
        z = jnp.dot(t, L2_ref[...], preferred_element_type=jnp.float32)
        z = z + lb2_ref[...]
        m = jnp.max(z, axis=1, keepdims=True)
        lse = jnp.log(jnp.sum(jnp.exp(z - m), axis=1, keepdims=True)) + m
        o_ref[...] = z - lse


_pool = pl.pallas_call(
    _pool_body,
    grid=(NB,),
    in_specs=[_rows2,
              pl.BlockSpec((1, 1, BN), lambda i: (i, 0, 0)),
              _wspec, _bspec,
              pl.BlockSpec((D, 128), _full),
              pl.BlockSpec((1, 128), _full)],
    out_specs=pl.BlockSpec((G, 128), _full),
    out_shape=jax.ShapeDtypeStruct((G, 128), jnp.float32),
    scratch_shapes=[pltpu.VMEM((G, D), jnp.float32)],
    compiler_params=pltpu.CompilerParams(
        dimension_semantics=("arbitrary",)),
)


def kernel(x, edge_index, batch,
           conv1_W1, conv1_b1, conv1_g, conv1_bb, conv1_W2, conv1_b2,
           conv2_W1, conv2_b1, conv2_g, conv2_bb, conv2_W2, conv2_b2,
           conv3_W1, conv3_b1, conv3_g, conv3_bb, conv3_W2, conv3_b2,
           conv4_W1, conv4_b1, conv4_g, conv4_bb, conv4_W2, conv4_b2,
           conv5_W1, conv5_b1, conv5_g, conv5_bb, conv5_W2, conv5_b2,
           lin1_W, lin1_b, lin2_W, lin2_b):
    f32 = jnp.float32
    x_pad = jnp.concatenate(
        [x, jnp.zeros((NPAD - N, F), f32)], axis=0)

    src, dst = lax.sort_key_val(edge_index[0], edge_index[1])
    pad = E_PAD - E
    src_p = jnp.concatenate([src, jnp.zeros((pad,), jnp.int32)])
    dst_p = jnp.concatenate([dst, jnp.full((pad,), DUMP, jnp.int32)])
    assert E_PAD == 32 * TR_E * 128 == 16 * TR_F * 128
    src_e = src_p.reshape(32, TR_E, 128)
    dst_e = dst_p.reshape(32, TR_E, 128)
    src16 = src_p.reshape(16, TR_F, 128)
    dst16 = dst_p.reshape(16, TR_F, 128)
    src_f = jnp.concatenate([src16, src16 + NPAD], axis=0)
    dst_f = jnp.concatenate([dst16, dst16], axis=0)
    zeros128 = jnp.zeros((128, HALF), f32)

    inv = 1.0 / jnp.sqrt(jnp.float32(1.0 + 1e-5))

    def fold(W1, b1, g, bb):
        gs = g * inv
        return W1 * gs[None, :], (b1 * gs + bb)[None, :]

    params = [
        fold(conv1_W1, conv1_b1, conv1_g, conv1_bb) + (conv1_W2, conv1_b2[None, :]),
        fold(conv2_W1, conv2_b1, conv2_g, conv2_bb) + (conv2_W2, conv2_b2[None, :]),
        fold(conv3_W1, conv3_b1, conv3_g, conv3_bb) + (conv3_W2, conv3_b2[None, :]),
        fold(conv4_W1, conv4_b1, conv4_g, conv4_bb) + (conv4_W2, conv4_b2[None, :]),
        fold(conv5_W1, conv5_b1, conv5_g, conv5_bb) + (conv5_W2, conv5_b2[None, :]),
    ]

    sc_layer1, sc_layerN = _get_sc_kernels()

    p1 = sc_layer1(x_pad, src_e, dst_e, zeros128)
    A, cvec, W2, b2 = params[0]
    h = _mlp1(x_pad, p1, A, cvec, W2, b2)

    for li in range(1, 5):
        a = sc_layerN(h.reshape(2 * NPAD, HALF), src_f, dst_f, zeros128)
        A, cvec, W2, b2 = params[li]
        h = _mlpN(h, a, A, cvec, W2, b2)

    batch_p = jnp.concatenate(
        [batch, jnp.full((NPAD - N,), G, jnp.int32)]).reshape(NB, 1, BN)
    L2p = jnp.concatenate(
        [lin2_W, jnp.zeros((D, 128 - C), f32)], axis=1)
    lb2p = jnp.concatenate(
        [lin2_b, jnp.full((128 - C,), -1e30, f32)])[None, :]
    out = _pool(h, batch_p, lin1_W, lin1_b[None, :], L2p, lb2p)
    return out[:, :C]


# speedup vs baseline: 1.4797x; 1.4797x over previous
import functools

import jax
import jax.numpy as jnp
from jax import lax
from jax.experimental import pallas as pl
from jax.experimental.pallas import tpu as pltpu
from jax.experimental.pallas import tpu_sc as plsc

N = 10000
E = 320000
F = 128
D = 256
C = 10
G = 128

NPAD = 10240
HALF = 128
E_PAD = 327680
R = 64
TR_E = 160
TR_F = 320
CH_F = 64
NCH_F = TR_F // CH_F
CH_E = 32
NCH_E = TR_E // CH_E
ROWS_PER_TILE = NPAD // 16
ZCHUNKS = ROWS_PER_TILE // R
DUMP = N

def _make_sc_body(ch, nch):
    def body(tab_hbm, src_hbm, dst_hbm, z_hbm, out_hbm,
             src_v, dst_v, b0, b1, b2, b3, acc,
             sg0, sg1, sg2, sg3, ss0, ss1, ss2, ss3):
        c = lax.axis_index("c")
        s = lax.axis_index("s")
        wid = c * 16 + s
        bufs = [b0, b1, b2, b3]
        sgs = [sg0, sg1, sg2, sg3]
        sss = [ss0, ss1, ss2, ss3]

        def wait(sem, buf):
            pltpu.make_async_copy(z_hbm, buf, sem).wait()

        pltpu.sync_copy(z_hbm, b0)
        for k in range(ZCHUNKS):
            pltpu.sync_copy(b0, acc.at[pl.ds((s * ZCHUNKS + k) * R, R)])
        plsc.subcore_barrier()

        def chunk(ci, carry):
            pltpu.sync_copy(src_hbm.at[wid, pl.ds(ci * ch, ch)], src_v)
            pltpu.sync_copy(dst_hbm.at[wid, pl.ds(ci * ch, ch)], dst_v)
            for k in range(3):
                pltpu.async_copy(tab_hbm.at[src_v.at[k]], bufs[k], sgs[k])

            def quad(jq, carry2):
                for k in range(4):
                    j = 4 * jq + k
                    km = (k + 3) % 4
                    wait(sgs[k], bufs[k])
                    pltpu.async_copy(bufs[k], acc.at[dst_v.at[j]],
                                     sss[k], add=True)

                    @pl.when(jnp.logical_and(j >= 1, j + 3 < ch))
                    def _():
                        wait(sss[km], bufs[km])

                    @pl.when(j + 3 < ch)
                    def _():
                        pltpu.async_copy(tab_hbm.at[src_v.at[j + 3]],
                                         bufs[km], sgs[km])

                return carry2

            lax.fori_loop(0, ch // 4, quad, carry, unroll=False)
            for k in range(4):
                wait(sss[k], bufs[k])
            return carry

        lax.fori_loop(0, nch, chunk, 0, unroll=False)
        plsc.subcore_barrier()
        pltpu.sync_copy(acc.at[pl.ds(s * ROWS_PER_TILE, ROWS_PER_TILE)],
                        out_hbm.at[c, pl.ds(s * ROWS_PER_TILE, ROWS_PER_TILE)])

    return body


@functools.lru_cache(maxsize=1)
def _get_sc_kernels():
    mesh = plsc.VectorSubcoreMesh(core_axis_name="c", subcore_axis_name="s",
                                  num_cores=2, num_subcores=16)

    def build(ch, nch):
        return pl.kernel(
            _make_sc_body(ch, nch),
            out_type=jax.ShapeDtypeStruct((2, NPAD, HALF), jnp.float32),
            mesh=mesh,
            scratch_types=[
                pltpu.VMEM((ch, R), jnp.int32),
                pltpu.VMEM((ch, R), jnp.int32),
                pltpu.VMEM((R, HALF), jnp.float32),
                pltpu.VMEM((R, HALF), jnp.float32),
                pltpu.VMEM((R, HALF), jnp.float32),
                pltpu.VMEM((R, HALF), jnp.float32),
                pltpu.VMEM_SHARED((NPAD, HALF), jnp.float32),
                pltpu.SemaphoreType.DMA,
                pltpu.SemaphoreType.DMA,
                pltpu.SemaphoreType.DMA,
                pltpu.SemaphoreType.DMA,
                pltpu.SemaphoreType.DMA,
                pltpu.SemaphoreType.DMA,
                pltpu.SemaphoreType.DMA,
                pltpu.SemaphoreType.DMA,
            ],
        )

    return build(CH_E, NCH_E), build(CH_F, NCH_F)


BN = 1024
NB = NPAD // BN


def _mlp1_body(x_ref, p_ref, A_ref, c_ref, W2_ref, b2_ref, o_ref):
    u = x_ref[...] + p_ref[0] + p_ref[1]
    y = jnp.dot(u, A_ref[...], preferred_element_type=jnp.float32) + c_ref[...]
    y = jnp.maximum(y, 0.0)
    z = jnp.dot(y, W2_ref[...], preferred_element_type=jnp.float32) + b2_ref[...]
    z = jnp.maximum(z, 0.0)
    o_ref[0] = z[:, :HALF]
    o_ref[1] = z[:, HALF:]


def _mlpN_body(h_ref, a_ref, A_ref, c_ref, W2_ref, b2_ref, o_ref):
    u = jnp.concatenate([h_ref[0] + a_ref[0], h_ref[1] + a_ref[1]], axis=1)
    y = jnp.dot(u, A_ref[...], preferred_element_type=jnp.float32) + c_ref[...]
    y = jnp.maximum(y, 0.0)
    z = jnp.dot(y, W2_ref[...], preferred_element_type=jnp.float32) + b2_ref[...]
    z = jnp.maximum(z, 0.0)
    o_ref[0] = z[:, :HALF]
    o_ref[1] = z[:, HALF:]


_full = lambda i: (0, 0)
_rows1 = pl.BlockSpec((BN, HALF), lambda i: (i, 0))
_rows2 = pl.BlockSpec((2, BN, HALF), lambda i: (0, i, 0))
_wspec = pl.BlockSpec((D, D), _full)
_w1spec = pl.BlockSpec((F, D), _full)
_bspec = pl.BlockSpec((1, D), _full)

_mlp1 = pl.pallas_call(
    _mlp1_body,
    grid=(NB,),
    in_specs=[_rows1, _rows2, _w1spec, _bspec, _wspec, _bspec],
    out_specs=_rows2,
    out_shape=jax.ShapeDtypeStruct((2, NPAD, HALF), jnp.float32),
    compiler_params=pltpu.CompilerParams(
        dimension_semantics=("parallel",)),
)

_mlpN = pl.pallas_call(
    _mlpN_body,
    grid=(NB,),
    in_specs=[_rows2, _rows2, _wspec, _bspec, _wspec, _bspec],
    out_specs=_rows2,
    out_shape=jax.ShapeDtypeStruct((2, NPAD, HALF), jnp.float32),
    compiler_params=pltpu.CompilerParams(
        dimension_semantics=("parallel",)),
)


def _pool_body(h_ref, b_ref, L1_ref, lb1_ref, L2_ref, lb2_ref, o_ref, hg_acc):
    i = pl.program_id(0)

    @pl.when(i == 0)
    def _():
        hg_acc[...] = jnp.zeros_like(hg_acc)

    b = b_ref[0, 0, :]
    oneh = (b[None, :] == lax.broadcasted_iota(jnp.int32, (G, BN), 0)
            ).astype(jnp.float32)
    hrow = jnp.concatenate([h_ref[0], h_ref[1]], axis=1)
    hg_acc[...] += jnp.dot(oneh, hrow, preferred_element_type=jnp.float32)

    @pl.when(i == NB - 1)
    def _():
        hg = hg_acc[...]
        t = jnp.dot(hg, L1_ref[...], preferred_element_type=jnp.float32)
        t = jnp.maximum(t + lb1_ref[...], 0.0)
        z = jnp.dot(t, L2_ref[...], preferred_element_type=jnp.float32)
        z = z + lb2_ref[...]
        m = jnp.max(z, axis=1, keepdims=True)
        lse = jnp.log(jnp.sum(jnp.exp(z - m), axis=1, keepdims=True)) + m
        o_ref[...] = z - lse


_pool = pl.pallas_call(
    _pool_body,
    grid=(NB,),
    in_specs=[_rows2,
              pl.BlockSpec((1, 1, BN), lambda i: (i, 0, 0)),
              _wspec, _bspec,
              pl.BlockSpec((D, 128), _full),
              pl.BlockSpec((1, 128), _full)],
    out_specs=pl.BlockSpec((G, 128), _full),
    out_shape=jax.ShapeDtypeStruct((G, 128), jnp.float32),
    scratch_shapes=[pltpu.VMEM((G, D), jnp.float32)],
    compiler_params=pltpu.CompilerParams(
        dimension_semantics=("arbitrary",)),
)


def kernel(x, edge_index, batch,
           conv1_W1, conv1_b1, conv1_g, conv1_bb, conv1_W2, conv1_b2,
           conv2_W1, conv2_b1, conv2_g, conv2_bb, conv2_W2, conv2_b2,
           conv3_W1, conv3_b1, conv3_g, conv3_bb, conv3_W2, conv3_b2,
           conv4_W1, conv4_b1, conv4_g, conv4_bb, conv4_W2, conv4_b2,
           conv5_W1, conv5_b1, conv5_g, conv5_bb, conv5_W2, conv5_b2,
           lin1_W, lin1_b, lin2_W, lin2_b):
    f32 = jnp.float32
    x_pad = jnp.concatenate(
        [x, jnp.zeros((NPAD - N, F), f32)], axis=0)

    src = edge_index[0]
    dst = edge_index[1]
    pad = E_PAD - E
    src_p = jnp.concatenate([src, jnp.zeros((pad,), jnp.int32)])
    dst_p = jnp.concatenate([dst, jnp.full((pad,), DUMP, jnp.int32)])
    assert E_PAD == 32 * TR_E * R == 16 * TR_F * R
    src_e = src_p.reshape(32, TR_E, R)
    dst_e = dst_p.reshape(32, TR_E, R)
    src16 = src_p.reshape(16, TR_F, R)
    dst16 = dst_p.reshape(16, TR_F, R)
    src_f = jnp.concatenate([src16, src16 + NPAD], axis=0)
    dst_f = jnp.concatenate([dst16, dst16], axis=0)
    zeros128 = jnp.zeros((R, HALF), f32)

    inv = 1.0 / jnp.sqrt(jnp.float32(1.0 + 1e-5))

    def fold(W1, b1, g, bb):
        gs = g * inv
        return W1 * gs[None, :], (b1 * gs + bb)[None, :]

    params = [
        fold(conv1_W1, conv1_b1, conv1_g, conv1_bb) + (conv1_W2, conv1_b2[None, :]),
        fold(conv2_W1, conv2_b1, conv2_g, conv2_bb) + (conv2_W2, conv2_b2[None, :]),
        fold(conv3_W1, conv3_b1, conv3_g, conv3_bb) + (conv3_W2, conv3_b2[None, :]),
        fold(conv4_W1, conv4_b1, conv4_g, conv4_bb) + (conv4_W2, conv4_b2[None, :]),
        fold(conv5_W1, conv5_b1, conv5_g, conv5_bb) + (conv5_W2, conv5_b2[None, :]),
    ]

    sc_layer1, sc_layerN = _get_sc_kernels()

    p1 = sc_layer1(x_pad, src_e, dst_e, zeros128)
    A, cvec, W2, b2 = params[0]
    h = _mlp1(x_pad, p1, A, cvec, W2, b2)

    for li in range(1, 5):
        a = sc_layerN(h.reshape(2 * NPAD, HALF), src_f, dst_f, zeros128)
        A, cvec, W2, b2 = params[li]
        h = _mlpN(h, a, A, cvec, W2, b2)

    batch_p = jnp.concatenate(
        [batch, jnp.full((NPAD - N,), G, jnp.int32)]).reshape(NB, 1, BN)
    L2p = jnp.concatenate(
        [lin2_W, jnp.zeros((D, 128 - C), f32)], axis=1)
    lb2p = jnp.concatenate(
        [lin2_b, jnp.full((128 - C,), -1e30, f32)])[None, :]
    out = _pool(h, batch_p, lin1_W, lin1_b[None, :], L2p, lb2p)
    return out[:, :C]
